# baseline (speedup 1.0000x reference)
import functools

import jax
import jax.numpy as jnp
from jax import lax
from jax.experimental import pallas as pl
from jax.experimental.pallas import tpu as pltpu
from jax.experimental.pallas import tpu_sc as plsc

_TOP_K = 8
_N_EXPERTS = 64
_HIDDEN = 4096
_ROWS = 8192

_NUM_CORES = 2
_NUM_SUBCORES = 16
_LANES = 16
_NW = _NUM_CORES * _NUM_SUBCORES
_RPT = _ROWS // _NW
_GROUPS = _RPT // _LANES

_BM = 512


def _gate_matmul_body(x_ref, w_ref, out_ref):
    out_ref[...] = lax.dot_general(
        x_ref[...], w_ref[...],
        dimension_numbers=(((1,), (1,)), ((), ())),
        preferred_element_type=jnp.float32,
    )


_gate_matmul = pl.pallas_call(
    _gate_matmul_body,
    grid=(_ROWS // _BM,),
    in_specs=[
        pl.BlockSpec((_BM, _HIDDEN), lambda i: (i, 0)),
        pl.BlockSpec((_N_EXPERTS, _HIDDEN), lambda i: (0, 0)),
    ],
    out_specs=pl.BlockSpec((_BM, _N_EXPERTS), lambda i: (i, 0)),
    out_shape=jax.ShapeDtypeStruct((_ROWS, _N_EXPERTS), jnp.float32),
)


_sc_mesh = plsc.VectorSubcoreMesh(
    core_axis_name="c", subcore_axis_name="s",
    num_cores=_NUM_CORES, num_subcores=_NUM_SUBCORES,
)


@functools.partial(
    pl.kernel,
    out_type=[
        jax.ShapeDtypeStruct((_ROWS * _TOP_K,), jnp.int32),
        jax.ShapeDtypeStruct((_ROWS * _TOP_K,), jnp.float32),
    ],
    mesh=_sc_mesh,
    compiler_params=pltpu.CompilerParams(
        use_tc_tiling_on_sc=False, needs_layout_passes=False),
    scratch_types=[
        pltpu.VMEM((_RPT * _N_EXPERTS,), jnp.float32),
        pltpu.VMEM((_RPT * _TOP_K,), jnp.int32),
        pltpu.VMEM((_RPT * _TOP_K,), jnp.float32),
    ],
)
def _sc_topk(logits_hbm, idx_hbm, w_hbm, lv, iv, wv):
    wid = lax.axis_index("s") * _NUM_CORES + lax.axis_index("c")
    base = wid * _RPT
    pltpu.sync_copy(logits_hbm.at[pl.ds(base * _N_EXPERTS, _RPT * _N_EXPERTS)], lv)

    lanes = lax.iota(jnp.int32, _LANES)
    neg_inf = jnp.full((_LANES,), -jnp.inf, jnp.float32)

    def group_body(g, carry):
        rows = g * _LANES + lanes
        rbase = rows * _N_EXPERTS
        obase = rows * _TOP_K
        vals, idxs = [], []
        for j in range(_TOP_K):
            bv = neg_inf
            bi = jnp.zeros((_LANES,), jnp.int32)
            for e in range(_N_EXPERTS):
                col = jnp.full((_LANES,), e, jnp.int32)
                v = plsc.load_gather(lv, [rbase + e])
                m = v > bv
                bv = jnp.where(m, v, bv)
                bi = jnp.where(m, col, bi)
            vals.append(bv)
            idxs.append(bi)
            if j < _TOP_K - 1:
                plsc.store_scatter(lv, [rbase + bi], neg_inf)
        top = vals[0]
        ws = [jnp.exp(v - top) for v in vals]
        denom = ws[0]
        for wj in ws[1:]:
            denom = denom + wj
        inv = 1.0 / denom
        for j in range(_TOP_K):
            plsc.store_scatter(iv, [obase + j], idxs[j])
            plsc.store_scatter(wv, [obase + j], ws[j] * inv)
        return carry

    lax.fori_loop(0, _GROUPS, group_body, 0)
    pltpu.sync_copy(iv, idx_hbm.at[pl.ds(base * _TOP_K, _RPT * _TOP_K)])
    pltpu.sync_copy(wv, w_hbm.at[pl.ds(base * _TOP_K, _RPT * _TOP_K)])


def kernel(hidden_states, weight):
    x = hidden_states.reshape(-1, _HIDDEN)
    logits = _gate_matmul(x, weight)
    idx_flat, w_flat = _sc_topk(logits.reshape(-1))
    return (idx_flat.reshape(_ROWS, _TOP_K), w_flat.reshape(_ROWS, _TOP_K))
